# Initial kernel scaffold; baseline (speedup 1.0000x reference)
import functools

import jax
import jax.numpy as jnp
from jax.experimental import pallas as pl
from jax.experimental.pallas import tpu as pltpu


def _erf(x):
    return jax.lax.erf(x)


def _gelu_exact(x):
    return x * 0.5 * (1.0 + _erf(x * 0.7071067811865476))


def _silu(x):
    return x * jax.nn.sigmoid(x)


def _router_body(x_ref, wr1_ref, br1_ref, g_ref, b_ref, wr2_ref, br2_ref,
                 probs_ref):
    x = x_ref[...]
    h = jnp.dot(x, wr1_ref[...], preferred_element_type=jnp.float32)
    h = h + br1_ref[...]
    mu = jnp.mean(h, axis=-1, keepdims=True)
    var = jnp.mean((h - mu) ** 2, axis=-1, keepdims=True)
    hn = (h - mu) * jax.lax.rsqrt(var + 1e-5) * g_ref[...] + b_ref[...]
    hg = _gelu_exact(hn)
    logits = jnp.dot(hg, wr2_ref[...], preferred_element_type=jnp.float32)
    logits = logits + br2_ref[...]
    m = jnp.max(logits, axis=-1, keepdims=True)
    ex = jnp.exp(logits - m)
    sm = ex / jnp.sum(ex, axis=-1, keepdims=True)
    lanes = jax.lax.broadcasted_iota(jnp.int32, sm.shape, 1)
    big = jnp.int32(1 << 20)
    v1 = jnp.max(sm, axis=-1, keepdims=True)
    i1 = jnp.min(jnp.where(sm == v1, lanes, big), axis=-1, keepdims=True)
    sm2 = jnp.where(lanes == i1, -1.0, sm)
    v2 = jnp.max(sm2, axis=-1, keepdims=True)
    i2 = jnp.min(jnp.where(sm2 == v2, lanes, big), axis=-1, keepdims=True)
    probs = jnp.where(lanes == i1, v1, 0.0) + jnp.where(lanes == i2, v2, 0.0)
    probs_ref[...] = probs


def _experts_body(x_ref, w1_ref, b1_ref, w2_ref, b2_ref, probs_ref, out_ref):
    e = pl.program_id(1)
    x = x_ref[...]
    h1 = jnp.dot(x, w1_ref[0], preferred_element_type=jnp.float32)
    h1 = h1 + b1_ref[...]
    a = jnp.where(e % 2 == 0, _gelu_exact(h1), _silu(h1))
    y = jnp.dot(a, w2_ref[0], preferred_element_type=jnp.float32)
    y = y + b2_ref[...]
    probs = probs_ref[...]
    lanes = jax.lax.broadcasted_iota(jnp.int32, probs.shape, 1)
    w = jnp.sum(jnp.where(lanes == e, probs, 0.0), axis=-1, keepdims=True)

    @pl.when(e == 0)
    def _():
        out_ref[...] = jnp.zeros_like(out_ref)

    out_ref[...] += w * y


def kernel(x, W1, b1, W2, b2, Wr1, br1, ln_g, ln_b, Wr2, br2, temp, bias):
    Bx, Lx, D = x.shape
    N = Bx * Lx
    E, _, DFF = W1.shape
    D2 = Wr1.shape[1]
    xf = x.reshape(N, D)

    wr2 = Wr2 / temp[0]
    br2f = (br2 / temp[0] + bias).reshape(1, E)

    BR = 2048
    probs = pl.pallas_call(
        _router_body,
        grid=(N // BR,),
        in_specs=[
            pl.BlockSpec((BR, D), lambda i: (i, 0)),
            pl.BlockSpec((D, D2), lambda i: (0, 0)),
            pl.BlockSpec((1, D2), lambda i: (0, 0)),
            pl.BlockSpec((1, D2), lambda i: (0, 0)),
            pl.BlockSpec((1, D2), lambda i: (0, 0)),
            pl.BlockSpec((D2, E), lambda i: (0, 0)),
            pl.BlockSpec((1, E), lambda i: (0, 0)),
        ],
        out_specs=pl.BlockSpec((BR, E), lambda i: (i, 0)),
        out_shape=jax.ShapeDtypeStruct((N, E), jnp.float32),
    )(xf, Wr1, br1.reshape(1, D2), ln_g.reshape(1, D2),
      ln_b.reshape(1, D2), wr2, br2f)

    BM = 512
    out = pl.pallas_call(
        _experts_body,
        grid=(N // BM, E),
        in_specs=[
            pl.BlockSpec((BM, D), lambda i, e: (i, 0)),
            pl.BlockSpec((1, D, DFF), lambda i, e: (e, 0, 0)),
            pl.BlockSpec((1, DFF), lambda i, e: (e, 0)),
            pl.BlockSpec((1, DFF, D), lambda i, e: (e, 0, 0)),
            pl.BlockSpec((1, D), lambda i, e: (e, 0)),
            pl.BlockSpec((BM, E), lambda i, e: (i, 0)),
        ],
        out_specs=pl.BlockSpec((BM, D), lambda i, e: (i, 0)),
        out_shape=jax.ShapeDtypeStruct((N, D), jnp.float32),
        compiler_params=pltpu.CompilerParams(
            dimension_semantics=("parallel", "arbitrary"),
        ),
    )(xf, W1, b1, W2, b2, probs)

    return out.reshape(Bx, Lx, D)


# speedup vs baseline: 1.9318x; 1.9318x over previous
import functools

import jax
import jax.numpy as jnp
from jax.experimental import pallas as pl
from jax.experimental.pallas import tpu as pltpu


def _erf(x):
    return jax.lax.erf(x)


def _gelu_exact(x):
    return x * 0.5 * (1.0 + _erf(x * 0.7071067811865476))


def _silu(x):
    return x * jax.nn.sigmoid(x)


def _router_body(x_ref, wr1_ref, br1_ref, g_ref, b_ref, wr2_ref, br2_ref,
                 probs_ref):
    x = x_ref[...]
    h = jnp.dot(x, wr1_ref[...], preferred_element_type=jnp.float32)
    h = h + br1_ref[...]
    mu = jnp.mean(h, axis=-1, keepdims=True)
    var = jnp.mean((h - mu) ** 2, axis=-1, keepdims=True)
    hn = (h - mu) * jax.lax.rsqrt(var + 1e-5) * g_ref[...] + b_ref[...]
    hg = _gelu_exact(hn)
    logits = jnp.dot(hg, wr2_ref[...], preferred_element_type=jnp.float32)
    logits = logits + br2_ref[...]
    m = jnp.max(logits, axis=-1, keepdims=True)
    ex = jnp.exp(logits - m)
    sm = ex / jnp.sum(ex, axis=-1, keepdims=True)
    lanes = jax.lax.broadcasted_iota(jnp.int32, sm.shape, 1)
    big = jnp.int32(1 << 20)
    v1 = jnp.max(sm, axis=-1, keepdims=True)
    i1 = jnp.min(jnp.where(sm == v1, lanes, big), axis=-1, keepdims=True)
    sm2 = jnp.where(lanes == i1, -1.0, sm)
    v2 = jnp.max(sm2, axis=-1, keepdims=True)
    i2 = jnp.min(jnp.where(sm2 == v2, lanes, big), axis=-1, keepdims=True)
    probs = jnp.where(lanes == i1, v1, 0.0) + jnp.where(lanes == i2, v2, 0.0)
    probs_ref[...] = probs


def _experts_body(x_ref, w1_ref, b1_ref, w2_ref, b2_ref, probs_ref, out_ref):
    e = pl.program_id(1)
    x = x_ref[...]
    h1 = jnp.dot(x, w1_ref[0], preferred_element_type=jnp.float32)
    h1 = h1 + b1_ref[0]
    a = jnp.where(e % 2 == 0, _gelu_exact(h1), _silu(h1))
    y = jnp.dot(a, w2_ref[0], preferred_element_type=jnp.float32)
    y = y + b2_ref[0]
    probs = probs_ref[...]
    lanes = jax.lax.broadcasted_iota(jnp.int32, probs.shape, 1)
    w = jnp.sum(jnp.where(lanes == e, probs, 0.0), axis=-1, keepdims=True)

    @pl.when(e == 0)
    def _():
        out_ref[...] = jnp.zeros_like(out_ref)

    out_ref[...] += w * y


def kernel(x, W1, b1, W2, b2, Wr1, br1, ln_g, ln_b, Wr2, br2, temp, bias):
    Bx, Lx, D = x.shape
    N = Bx * Lx
    E, _, DFF = W1.shape
    D2 = Wr1.shape[1]
    xf = x.reshape(N, D)

    wr2 = Wr2 / temp[0]
    br2f = (br2 / temp[0] + bias).reshape(1, E)

    BR = 2048
    probs = pl.pallas_call(
        _router_body,
        grid=(N // BR,),
        in_specs=[
            pl.BlockSpec((BR, D), lambda i: (i, 0)),
            pl.BlockSpec((D, D2), lambda i: (0, 0)),
            pl.BlockSpec((1, D2), lambda i: (0, 0)),
            pl.BlockSpec((1, D2), lambda i: (0, 0)),
            pl.BlockSpec((1, D2), lambda i: (0, 0)),
            pl.BlockSpec((D2, E), lambda i: (0, 0)),
            pl.BlockSpec((1, E), lambda i: (0, 0)),
        ],
        out_specs=pl.BlockSpec((BR, E), lambda i: (i, 0)),
        out_shape=jax.ShapeDtypeStruct((N, E), jnp.float32),
    )(xf, Wr1, br1.reshape(1, D2), ln_g.reshape(1, D2),
      ln_b.reshape(1, D2), wr2, br2f)

    BM = 512
    out = pl.pallas_call(
        _experts_body,
        grid=(N // BM, E),
        in_specs=[
            pl.BlockSpec((BM, D), lambda i, e: (i, 0)),
            pl.BlockSpec((1, D, DFF), lambda i, e: (e, 0, 0)),
            pl.BlockSpec((1, 1, DFF), lambda i, e: (e, 0, 0)),
            pl.BlockSpec((1, DFF, D), lambda i, e: (e, 0, 0)),
            pl.BlockSpec((1, 1, D), lambda i, e: (e, 0, 0)),
            pl.BlockSpec((BM, E), lambda i, e: (i, 0)),
        ],
        out_specs=pl.BlockSpec((BM, D), lambda i, e: (i, 0)),
        out_shape=jax.ShapeDtypeStruct((N, D), jnp.float32),
        compiler_params=pltpu.CompilerParams(
            dimension_semantics=("parallel", "arbitrary"),
        ),
    )(xf, W1, b1.reshape(E, 1, DFF), W2, b2.reshape(E, 1, D), probs)

    return out.reshape(Bx, Lx, D)
